# baseline (speedup 1.0000x reference)
import functools

import jax
import jax.numpy as jnp
from jax import lax
from jax.experimental import pallas as pl
from jax.experimental.pallas import tpu as pltpu
from jax.experimental.pallas import tpu_sc as plsc

N = 10000
E = 320000
F_IN = 128
H = 32
G = 64

NC, NS, L = 2, 16, 16
NW = NC * NS
CHUNK = 128
EPT_CHUNKS = 80
E_PAD = NW * EPT_CHUNKS * CHUNK
K = 5
NGRP = EPT_CHUNKS // K
N_PAD = 10240
RPS = N_PAD // NS
DF = 16

_mesh = plsc.VectorSubcoreMesh(
    core_axis_name="c", subcore_axis_name="s", num_cores=NC, num_subcores=NS)
_sc_params = pltpu.CompilerParams(use_tc_tiling_on_sc=False)


@functools.partial(
    pl.kernel,
    out_type=jax.ShapeDtypeStruct((NC, N_PAD, DF), jnp.float32),
    mesh=_mesh,
    scratch_types=[
        pltpu.VMEM((EPT_CHUNKS, CHUNK), jnp.int32),
        pltpu.VMEM((CHUNK, DF), jnp.float32),
        pltpu.VMEM_SHARED((N_PAD, DF), jnp.float32),
        pltpu.SemaphoreType.DMA,
    ],
    compiler_params=_sc_params,
)
def _deg_sc(dstT_hbm, out_hbm, dst_v, ones_v, acc_sh, sem):
    c = lax.axis_index("c")
    s = lax.axis_index("s")
    wid = s * NC + c
    pltpu.sync_copy(dstT_hbm.at[wid], dst_v)

    def _fill(val, i, carry):
        ones_v[i, pl.ds(0, 16)] = jnp.full((16,), val, jnp.float32)
        return carry

    lax.fori_loop(0, CHUNK, functools.partial(_fill, 0.0), 0)
    for k in range(RPS // CHUNK):
        pltpu.sync_copy(ones_v, acc_sh.at[pl.ds(s * RPS + k * CHUNK, CHUNK)])
    lax.fori_loop(0, CHUNK, functools.partial(_fill, 1.0), 0)
    plsc.subcore_barrier()

    def _body(g, carry):
        for b in range(8):
            pltpu.async_copy(ones_v, acc_sh.at[dst_v.at[g * 8 + b]], sem,
                             add=True)
        for b in range(8):
            pltpu.make_async_copy(ones_v, acc_sh.at[pl.ds(0, CHUNK)],
                                  sem).wait()
        return carry

    lax.fori_loop(0, EPT_CHUNKS // 8, _body, 0)
    plsc.subcore_barrier()
    pltpu.sync_copy(acc_sh.at[pl.ds(s * RPS, RPS)],
                    out_hbm.at[c, pl.ds(s * RPS, RPS)])


@functools.partial(
    pl.kernel,
    out_type=jax.ShapeDtypeStruct((NC, N_PAD, H), jnp.float32),
    mesh=_mesh,
    scratch_types=[
        pltpu.VMEM((EPT_CHUNKS, CHUNK), jnp.int32),
        pltpu.VMEM((EPT_CHUNKS, CHUNK), jnp.int32),
        pltpu.VMEM((2 * K, CHUNK, H), jnp.float32),
        pltpu.VMEM_SHARED((N_PAD, H), jnp.float32),
        pltpu.SemaphoreType.DMA,
        pltpu.SemaphoreType.DMA,
    ],
    compiler_params=_sc_params,
)
def _agg_sc(hs_hbm, srcT_hbm, dstT_hbm, out_hbm, src_v, dst_v, buf_v, acc_sh,
            gsem, ssem):
    c = lax.axis_index("c")
    s = lax.axis_index("s")
    wid = s * NC + c
    pltpu.sync_copy(srcT_hbm.at[wid], src_v)
    pltpu.sync_copy(dstT_hbm.at[wid], dst_v)

    def _zfill(i, carry):
        buf_v[0, i, pl.ds(0, 16)] = jnp.zeros((16,), jnp.float32)
        buf_v[0, i, pl.ds(16, 16)] = jnp.zeros((16,), jnp.float32)
        return carry

    lax.fori_loop(0, CHUNK, _zfill, 0)
    for k in range(RPS // CHUNK):
        pltpu.sync_copy(buf_v.at[0],
                        acc_sh.at[pl.ds(s * RPS + k * CHUNK, CHUNK)])
    plsc.subcore_barrier()

    for b in range(K):
        pltpu.async_copy(hs_hbm.at[src_v.at[b]], buf_v.at[b], gsem)

    def _body(g, carry):
        base = g * K
        boff = (g % 2) * K
        nboff = K - boff
        for b in range(K):
            pltpu.make_async_copy(hs_hbm.at[pl.ds(0, CHUNK)],
                                  buf_v.at[b], gsem).wait()
        for b in range(K):
            pltpu.async_copy(buf_v.at[boff + b],
                             acc_sh.at[dst_v.at[base + b]], ssem, add=True)

        @pl.when(g < NGRP - 1)
        def _():
            for b in range(K):
                pltpu.async_copy(hs_hbm.at[src_v.at[base + K + b]],
                                 buf_v.at[nboff + b], gsem)

        for b in range(K):
            pltpu.make_async_copy(buf_v.at[b], acc_sh.at[pl.ds(0, CHUNK)],
                                  ssem).wait()
        return carry

    lax.fori_loop(0, NGRP, _body, 0)
    plsc.subcore_barrier()
    pltpu.sync_copy(acc_sh.at[pl.ds(s * RPS, RPS)],
                    out_hbm.at[c, pl.ds(s * RPS, RPS)])


def _prep_body(x_ref, w_ref, degp_ref, hs_ref, dinv_ref):
    deg = degp_ref[0, :, 0:1] + degp_ref[1, :, 0:1] + 1.0
    dinv = lax.rsqrt(deg)
    h = jnp.dot(x_ref[...], w_ref[...], preferred_element_type=jnp.float32)
    hs_ref[...] = h * dinv
    dinv_ref[...] = dinv


def _prep_tc(x_pad, W1, deg_parts):
    return pl.pallas_call(
        _prep_body,
        out_shape=[
            jax.ShapeDtypeStruct((N_PAD, H), jnp.float32),
            jax.ShapeDtypeStruct((N_PAD, 1), jnp.float32),
        ],
    )(x_pad, W1, deg_parts)


def _mid_body(aggp_ref, hs_ref, dinv_ref, b_ref, g_ref, be_ref, wn_ref,
              out_ref):
    dinv = dinv_ref[...]
    pre = dinv * (aggp_ref[0] + aggp_ref[1] + hs_ref[...]) + b_ref[...]
    t = jnp.tanh(pre)
    mask = (lax.broadcasted_iota(jnp.int32, (N_PAD, 1), 0) < N).astype(
        jnp.float32)
    m = jnp.sum(t * mask, axis=0, keepdims=True) * (1.0 / N)
    d = (t - m) * mask
    var = jnp.sum(d * d, axis=0, keepdims=True) * (1.0 / N)
    bn = g_ref[...] * (t - m) * lax.rsqrt(var + 1e-5) + be_ref[...]
    hn = jnp.dot(bn, wn_ref[...], preferred_element_type=jnp.float32)
    out_ref[...] = hn * dinv * mask


def _mid_tc(aggp, hs, dinv, b, g, be, Wn):
    return pl.pallas_call(
        _mid_body,
        out_shape=jax.ShapeDtypeStruct((N_PAD, H), jnp.float32),
    )(aggp, hs, dinv, b.reshape(1, H), g.reshape(1, H), be.reshape(1, H), Wn)


def _final_body(aggp_ref, hs_ref, dinv_ref, b_ref, batch_ref, wc_ref, bc_ref,
                out_ref):
    dinv = dinv_ref[...]
    pre = dinv * (aggp_ref[0] + aggp_ref[1] + hs_ref[...]) + b_ref[...]
    t = jnp.tanh(pre)
    gid = lax.broadcasted_iota(jnp.int32, (1, G), 1)
    P = (batch_ref[...] == gid).astype(jnp.float32)
    sums = lax.dot_general(P, t, (((0,), (0,)), ((), ())),
                           preferred_element_type=jnp.float32)
    ones = jnp.ones((N_PAD, 1), jnp.float32)
    counts = lax.dot_general(P, ones, (((0,), (0,)), ((), ())),
                             preferred_element_type=jnp.float32)
    pooled = sums / jnp.maximum(counts, 1.0)
    out_ref[...] = jnp.dot(pooled, wc_ref[...],
                           preferred_element_type=jnp.float32) + bc_ref[...]


def _final_tc(aggp, hs, dinv, b, batch_pad, Wc, bc):
    return pl.pallas_call(
        _final_body,
        out_shape=jax.ShapeDtypeStruct((G, 1), jnp.float32),
    )(aggp, hs, dinv, b.reshape(1, H), batch_pad, Wc, bc.reshape(1, 1))


def kernel(x, edge_index, batch, W1, b1, g1, be1, W2, b2, g2, be2, W3, b3,
           Wc, bc):
    x_pad = jnp.pad(x, ((0, N_PAD - N), (0, 0)))
    pad_e = E_PAD - E
    src = jnp.concatenate(
        [edge_index[0], jnp.full((pad_e,), N, jnp.int32)])
    dst = jnp.concatenate(
        [edge_index[1], jnp.full((pad_e,), N, jnp.int32)])
    srcT = src.reshape(NW, EPT_CHUNKS, CHUNK)
    dstT = dst.reshape(NW, EPT_CHUNKS, CHUNK)
    batch_pad = jnp.pad(batch, (0, N_PAD - N),
                        constant_values=G).reshape(N_PAD, 1)

    deg_parts = _deg_sc(dstT)
    hs1, dinv = _prep_tc(x_pad, W1, deg_parts)
    agg1 = _agg_sc(hs1, srcT, dstT)
    hs2 = _mid_tc(agg1, hs1, dinv, b1, g1, be1, W2)
    agg2 = _agg_sc(hs2, srcT, dstT)
    hs3 = _mid_tc(agg2, hs2, dinv, b2, g2, be2, W3)
    agg3 = _agg_sc(hs3, srcT, dstT)
    out = _final_tc(agg3, hs3, dinv, b3, batch_pad, Wc, bc)
    return out


# speedup vs baseline: 1.9364x; 1.9364x over previous
import functools

import jax
import jax.numpy as jnp
from jax import lax
from jax.experimental import pallas as pl
from jax.experimental.pallas import tpu as pltpu
from jax.experimental.pallas import tpu_sc as plsc

N = 10000
E = 320000
F_IN = 128
H = 32
G = 64

NC, NS, L = 2, 16, 16
NW = NC * NS
CHUNK = 128
EPT_CHUNKS = 80
E_PAD = NW * EPT_CHUNKS * CHUNK
K = 5
NGRP = EPT_CHUNKS // K
N_PAD = 10240
RPS = N_PAD // NS
DF = 16

_mesh = plsc.VectorSubcoreMesh(
    core_axis_name="c", subcore_axis_name="s", num_cores=NC, num_subcores=NS)
_sc_params = pltpu.CompilerParams(use_tc_tiling_on_sc=False)


@functools.partial(
    pl.kernel,
    out_type=jax.ShapeDtypeStruct((NC, N_PAD, DF), jnp.float32),
    mesh=_mesh,
    scratch_types=[
        pltpu.VMEM((EPT_CHUNKS, CHUNK), jnp.int32),
        pltpu.VMEM((CHUNK, DF), jnp.float32),
        pltpu.VMEM_SHARED((N_PAD, DF), jnp.float32),
        pltpu.SemaphoreType.DMA,
    ],
    compiler_params=_sc_params,
)
def _deg_sc(dstT_hbm, out_hbm, dst_v, ones_v, acc_sh, sem):
    c = lax.axis_index("c")
    s = lax.axis_index("s")
    wid = s * NC + c
    pltpu.sync_copy(dstT_hbm.at[wid], dst_v)

    def _fill(val, i, carry):
        ones_v[i, pl.ds(0, 16)] = jnp.full((16,), val, jnp.float32)
        return carry

    lax.fori_loop(0, CHUNK, functools.partial(_fill, 0.0), 0)
    for k in range(RPS // CHUNK):
        pltpu.sync_copy(ones_v, acc_sh.at[pl.ds(s * RPS + k * CHUNK, CHUNK)])
    lax.fori_loop(0, CHUNK, functools.partial(_fill, 1.0), 0)
    plsc.subcore_barrier()

    def _body(g, carry):
        for b in range(8):
            pltpu.async_copy(ones_v, acc_sh.at[dst_v.at[g * 8 + b]], sem,
                             add=True)
        for b in range(8):
            pltpu.make_async_copy(ones_v, acc_sh.at[pl.ds(0, CHUNK)],
                                  sem).wait()
        return carry

    lax.fori_loop(0, EPT_CHUNKS // 8, _body, 0)
    plsc.subcore_barrier()
    pltpu.sync_copy(acc_sh.at[pl.ds(s * RPS, RPS)],
                    out_hbm.at[c, pl.ds(s * RPS, RPS)])


@functools.partial(
    pl.kernel,
    out_type=jax.ShapeDtypeStruct((NC, N_PAD, H), jnp.float32),
    mesh=_mesh,
    scratch_types=[
        pltpu.VMEM((EPT_CHUNKS, CHUNK), jnp.int32),
        pltpu.VMEM((EPT_CHUNKS, CHUNK), jnp.int32),
        pltpu.VMEM((2 * K, CHUNK, H), jnp.float32),
        pltpu.VMEM_SHARED((N_PAD, H), jnp.float32),
        pltpu.VMEM_SHARED((N_PAD, H), jnp.float32),
        pltpu.SemaphoreType.DMA,
        pltpu.SemaphoreType.DMA,
    ],
    compiler_params=_sc_params,
)
def _agg_sc(hs_hbm, srcT_hbm, dstT_hbm, out_hbm, src_v, dst_v, buf_v, acc_sh,
            hs_sh, gsem, ssem):
    c = lax.axis_index("c")
    s = lax.axis_index("s")
    wid = s * NC + c
    pltpu.sync_copy(srcT_hbm.at[wid], src_v)
    pltpu.sync_copy(dstT_hbm.at[wid], dst_v)
    pltpu.sync_copy(hs_hbm.at[pl.ds(s * RPS, RPS)],
                    hs_sh.at[pl.ds(s * RPS, RPS)])

    def _zfill(i, carry):
        buf_v[0, i, pl.ds(0, 16)] = jnp.zeros((16,), jnp.float32)
        buf_v[0, i, pl.ds(16, 16)] = jnp.zeros((16,), jnp.float32)
        return carry

    lax.fori_loop(0, CHUNK, _zfill, 0)
    for k in range(RPS // CHUNK):
        pltpu.sync_copy(buf_v.at[0],
                        acc_sh.at[pl.ds(s * RPS + k * CHUNK, CHUNK)])
    plsc.subcore_barrier()

    for b in range(K):
        pltpu.async_copy(hs_sh.at[src_v.at[b]], buf_v.at[b], gsem)

    def _body(g, carry):
        base = g * K
        boff = (g % 2) * K
        nboff = K - boff
        for b in range(K):
            pltpu.make_async_copy(hs_hbm.at[pl.ds(0, CHUNK)],
                                  buf_v.at[b], gsem).wait()
        for b in range(K):
            pltpu.async_copy(buf_v.at[boff + b],
                             acc_sh.at[dst_v.at[base + b]], ssem, add=True)

        @pl.when(g < NGRP - 1)
        def _():
            for b in range(K):
                pltpu.async_copy(hs_sh.at[src_v.at[base + K + b]],
                                 buf_v.at[nboff + b], gsem)

        for b in range(K):
            pltpu.make_async_copy(buf_v.at[b], acc_sh.at[pl.ds(0, CHUNK)],
                                  ssem).wait()
        return carry

    lax.fori_loop(0, NGRP, _body, 0)
    plsc.subcore_barrier()
    pltpu.sync_copy(acc_sh.at[pl.ds(s * RPS, RPS)],
                    out_hbm.at[c, pl.ds(s * RPS, RPS)])


def _prep_body(x_ref, w_ref, degp_ref, hs_ref, dinv_ref):
    deg = degp_ref[0, :, 0:1] + degp_ref[1, :, 0:1] + 1.0
    dinv = lax.rsqrt(deg)
    h = jnp.dot(x_ref[...], w_ref[...], preferred_element_type=jnp.float32)
    hs_ref[...] = h * dinv
    dinv_ref[...] = dinv


def _prep_tc(x_pad, W1, deg_parts):
    return pl.pallas_call(
        _prep_body,
        out_shape=[
            jax.ShapeDtypeStruct((N_PAD, H), jnp.float32),
            jax.ShapeDtypeStruct((N_PAD, 1), jnp.float32),
        ],
    )(x_pad, W1, deg_parts)


def _mid_body(aggp_ref, hs_ref, dinv_ref, b_ref, g_ref, be_ref, wn_ref,
              out_ref):
    dinv = dinv_ref[...]
    pre = dinv * (aggp_ref[0] + aggp_ref[1] + hs_ref[...]) + b_ref[...]
    t = jnp.tanh(pre)
    mask = (lax.broadcasted_iota(jnp.int32, (N_PAD, 1), 0) < N).astype(
        jnp.float32)
    m = jnp.sum(t * mask, axis=0, keepdims=True) * (1.0 / N)
    d = (t - m) * mask
    var = jnp.sum(d * d, axis=0, keepdims=True) * (1.0 / N)
    bn = g_ref[...] * (t - m) * lax.rsqrt(var + 1e-5) + be_ref[...]
    hn = jnp.dot(bn, wn_ref[...], preferred_element_type=jnp.float32)
    out_ref[...] = hn * dinv * mask


def _mid_tc(aggp, hs, dinv, b, g, be, Wn):
    return pl.pallas_call(
        _mid_body,
        out_shape=jax.ShapeDtypeStruct((N_PAD, H), jnp.float32),
    )(aggp, hs, dinv, b.reshape(1, H), g.reshape(1, H), be.reshape(1, H), Wn)


def _final_body(aggp_ref, hs_ref, dinv_ref, b_ref, batch_ref, wc_ref, bc_ref,
                out_ref):
    dinv = dinv_ref[...]
    pre = dinv * (aggp_ref[0] + aggp_ref[1] + hs_ref[...]) + b_ref[...]
    t = jnp.tanh(pre)
    gid = lax.broadcasted_iota(jnp.int32, (1, G), 1)
    P = (batch_ref[...] == gid).astype(jnp.float32)
    sums = lax.dot_general(P, t, (((0,), (0,)), ((), ())),
                           preferred_element_type=jnp.float32)
    ones = jnp.ones((N_PAD, 1), jnp.float32)
    counts = lax.dot_general(P, ones, (((0,), (0,)), ((), ())),
                             preferred_element_type=jnp.float32)
    pooled = sums / jnp.maximum(counts, 1.0)
    out_ref[...] = jnp.dot(pooled, wc_ref[...],
                           preferred_element_type=jnp.float32) + bc_ref[...]


def _final_tc(aggp, hs, dinv, b, batch_pad, Wc, bc):
    return pl.pallas_call(
        _final_body,
        out_shape=jax.ShapeDtypeStruct((G, 1), jnp.float32),
    )(aggp, hs, dinv, b.reshape(1, H), batch_pad, Wc, bc.reshape(1, 1))


def kernel(x, edge_index, batch, W1, b1, g1, be1, W2, b2, g2, be2, W3, b3,
           Wc, bc):
    x_pad = jnp.pad(x, ((0, N_PAD - N), (0, 0)))
    pad_e = E_PAD - E
    src = jnp.concatenate(
        [edge_index[0], jnp.full((pad_e,), N, jnp.int32)])
    dst = jnp.concatenate(
        [edge_index[1], jnp.full((pad_e,), N, jnp.int32)])
    srcT = src.reshape(NW, EPT_CHUNKS, CHUNK)
    dstT = dst.reshape(NW, EPT_CHUNKS, CHUNK)
    batch_pad = jnp.pad(batch, (0, N_PAD - N),
                        constant_values=G).reshape(N_PAD, 1)

    deg_parts = _deg_sc(dstT)
    hs1, dinv = _prep_tc(x_pad, W1, deg_parts)
    agg1 = _agg_sc(hs1, srcT, dstT)
    hs2 = _mid_tc(agg1, hs1, dinv, b1, g1, be1, W2)
    agg2 = _agg_sc(hs2, srcT, dstT)
    hs3 = _mid_tc(agg2, hs2, dinv, b2, g2, be2, W3)
    agg3 = _agg_sc(hs3, srcT, dstT)
    out = _final_tc(agg3, hs3, dinv, b3, batch_pad, Wc, bc)
    return out


# speedup vs baseline: 1.9447x; 1.0043x over previous
import functools

import jax
import jax.numpy as jnp
from jax import lax
from jax.experimental import pallas as pl
from jax.experimental.pallas import tpu as pltpu
from jax.experimental.pallas import tpu_sc as plsc

N = 10000
E = 320000
F_IN = 128
H = 32
G = 64

NC, NS, L = 2, 16, 16
NW = NC * NS
CHUNK = 128
EPT_CHUNKS = 80
E_PAD = NW * EPT_CHUNKS * CHUNK
K = 8
NGRP = EPT_CHUNKS // K
N_PAD = 10240
RPS = N_PAD // NS
DF = 16

_mesh = plsc.VectorSubcoreMesh(
    core_axis_name="c", subcore_axis_name="s", num_cores=NC, num_subcores=NS)
_sc_params = pltpu.CompilerParams(use_tc_tiling_on_sc=False)


@functools.partial(
    pl.kernel,
    out_type=jax.ShapeDtypeStruct((NC, N_PAD, DF), jnp.float32),
    mesh=_mesh,
    scratch_types=[
        pltpu.VMEM((EPT_CHUNKS, CHUNK), jnp.int32),
        pltpu.VMEM((CHUNK, DF), jnp.float32),
        pltpu.VMEM_SHARED((N_PAD, DF), jnp.float32),
        pltpu.SemaphoreType.DMA,
    ],
    compiler_params=_sc_params,
)
def _deg_sc(dstT_hbm, out_hbm, dst_v, ones_v, acc_sh, sem):
    c = lax.axis_index("c")
    s = lax.axis_index("s")
    wid = s * NC + c
    pltpu.sync_copy(dstT_hbm.at[wid], dst_v)

    def _fill(val, i, carry):
        ones_v[i, pl.ds(0, 16)] = jnp.full((16,), val, jnp.float32)
        return carry

    lax.fori_loop(0, CHUNK, functools.partial(_fill, 0.0), 0)
    for k in range(RPS // CHUNK):
        pltpu.sync_copy(ones_v, acc_sh.at[pl.ds(s * RPS + k * CHUNK, CHUNK)])
    lax.fori_loop(0, CHUNK, functools.partial(_fill, 1.0), 0)
    plsc.subcore_barrier()

    def _body(g, carry):
        for b in range(8):
            pltpu.async_copy(ones_v, acc_sh.at[dst_v.at[g * 8 + b]], sem,
                             add=True)
        for b in range(8):
            pltpu.make_async_copy(ones_v, acc_sh.at[pl.ds(0, CHUNK)],
                                  sem).wait()
        return carry

    lax.fori_loop(0, EPT_CHUNKS // 8, _body, 0)
    plsc.subcore_barrier()
    pltpu.sync_copy(acc_sh.at[pl.ds(s * RPS, RPS)],
                    out_hbm.at[c, pl.ds(s * RPS, RPS)])


@functools.partial(
    pl.kernel,
    out_type=jax.ShapeDtypeStruct((NC, N_PAD, H), jnp.float32),
    mesh=_mesh,
    scratch_types=[
        pltpu.VMEM((EPT_CHUNKS, CHUNK), jnp.int32),
        pltpu.VMEM((EPT_CHUNKS, CHUNK), jnp.int32),
        pltpu.VMEM((2 * K, CHUNK, H), jnp.float32),
        pltpu.VMEM_SHARED((N_PAD, H), jnp.float32),
        pltpu.VMEM_SHARED((N_PAD, H), jnp.float32),
        pltpu.SemaphoreType.DMA,
        pltpu.SemaphoreType.DMA,
    ],
    compiler_params=_sc_params,
)
def _agg_sc(hs_hbm, srcT_hbm, dstT_hbm, out_hbm, src_v, dst_v, buf_v, acc_sh,
            hs_sh, gsem, ssem):
    c = lax.axis_index("c")
    s = lax.axis_index("s")
    wid = s * NC + c
    pltpu.sync_copy(srcT_hbm.at[wid], src_v)
    pltpu.sync_copy(dstT_hbm.at[wid], dst_v)
    pltpu.sync_copy(hs_hbm.at[pl.ds(s * RPS, RPS)],
                    hs_sh.at[pl.ds(s * RPS, RPS)])

    def _zfill(i, carry):
        buf_v[0, i, pl.ds(0, 16)] = jnp.zeros((16,), jnp.float32)
        buf_v[0, i, pl.ds(16, 16)] = jnp.zeros((16,), jnp.float32)
        return carry

    lax.fori_loop(0, CHUNK, _zfill, 0)
    for k in range(RPS // CHUNK):
        pltpu.sync_copy(buf_v.at[0],
                        acc_sh.at[pl.ds(s * RPS + k * CHUNK, CHUNK)])
    plsc.subcore_barrier()

    for b in range(K):
        pltpu.async_copy(hs_sh.at[src_v.at[b]], buf_v.at[b], gsem)

    def _body(g, carry):
        base = g * K
        boff = (g % 2) * K
        nboff = K - boff
        for b in range(K):
            pltpu.make_async_copy(hs_hbm.at[pl.ds(0, CHUNK)],
                                  buf_v.at[b], gsem).wait()
        for b in range(K):
            pltpu.async_copy(buf_v.at[boff + b],
                             acc_sh.at[dst_v.at[base + b]], ssem, add=True)

        @pl.when(g < NGRP - 1)
        def _():
            for b in range(K):
                pltpu.async_copy(hs_sh.at[src_v.at[base + K + b]],
                                 buf_v.at[nboff + b], gsem)

        for b in range(K):
            pltpu.make_async_copy(buf_v.at[b], acc_sh.at[pl.ds(0, CHUNK)],
                                  ssem).wait()
        return carry

    lax.fori_loop(0, NGRP, _body, 0)
    plsc.subcore_barrier()
    pltpu.sync_copy(acc_sh.at[pl.ds(s * RPS, RPS)],
                    out_hbm.at[c, pl.ds(s * RPS, RPS)])


def _prep_body(x_ref, w_ref, degp_ref, hs_ref, dinv_ref):
    deg = degp_ref[0, :, 0:1] + degp_ref[1, :, 0:1] + 1.0
    dinv = lax.rsqrt(deg)
    h = jnp.dot(x_ref[...], w_ref[...], preferred_element_type=jnp.float32)
    hs_ref[...] = h * dinv
    dinv_ref[...] = dinv


def _prep_tc(x_pad, W1, deg_parts):
    return pl.pallas_call(
        _prep_body,
        out_shape=[
            jax.ShapeDtypeStruct((N_PAD, H), jnp.float32),
            jax.ShapeDtypeStruct((N_PAD, 1), jnp.float32),
        ],
    )(x_pad, W1, deg_parts)


def _mid_body(aggp_ref, hs_ref, dinv_ref, b_ref, g_ref, be_ref, wn_ref,
              out_ref):
    dinv = dinv_ref[...]
    pre = dinv * (aggp_ref[0] + aggp_ref[1] + hs_ref[...]) + b_ref[...]
    t = jnp.tanh(pre)
    mask = (lax.broadcasted_iota(jnp.int32, (N_PAD, 1), 0) < N).astype(
        jnp.float32)
    m = jnp.sum(t * mask, axis=0, keepdims=True) * (1.0 / N)
    d = (t - m) * mask
    var = jnp.sum(d * d, axis=0, keepdims=True) * (1.0 / N)
    bn = g_ref[...] * (t - m) * lax.rsqrt(var + 1e-5) + be_ref[...]
    hn = jnp.dot(bn, wn_ref[...], preferred_element_type=jnp.float32)
    out_ref[...] = hn * dinv * mask


def _mid_tc(aggp, hs, dinv, b, g, be, Wn):
    return pl.pallas_call(
        _mid_body,
        out_shape=jax.ShapeDtypeStruct((N_PAD, H), jnp.float32),
    )(aggp, hs, dinv, b.reshape(1, H), g.reshape(1, H), be.reshape(1, H), Wn)


def _final_body(aggp_ref, hs_ref, dinv_ref, b_ref, batch_ref, wc_ref, bc_ref,
                out_ref):
    dinv = dinv_ref[...]
    pre = dinv * (aggp_ref[0] + aggp_ref[1] + hs_ref[...]) + b_ref[...]
    t = jnp.tanh(pre)
    gid = lax.broadcasted_iota(jnp.int32, (1, G), 1)
    P = (batch_ref[...] == gid).astype(jnp.float32)
    sums = lax.dot_general(P, t, (((0,), (0,)), ((), ())),
                           preferred_element_type=jnp.float32)
    ones = jnp.ones((N_PAD, 1), jnp.float32)
    counts = lax.dot_general(P, ones, (((0,), (0,)), ((), ())),
                             preferred_element_type=jnp.float32)
    pooled = sums / jnp.maximum(counts, 1.0)
    out_ref[...] = jnp.dot(pooled, wc_ref[...],
                           preferred_element_type=jnp.float32) + bc_ref[...]


def _final_tc(aggp, hs, dinv, b, batch_pad, Wc, bc):
    return pl.pallas_call(
        _final_body,
        out_shape=jax.ShapeDtypeStruct((G, 1), jnp.float32),
    )(aggp, hs, dinv, b.reshape(1, H), batch_pad, Wc, bc.reshape(1, 1))


def kernel(x, edge_index, batch, W1, b1, g1, be1, W2, b2, g2, be2, W3, b3,
           Wc, bc):
    x_pad = jnp.pad(x, ((0, N_PAD - N), (0, 0)))
    pad_e = E_PAD - E
    src = jnp.concatenate(
        [edge_index[0], jnp.full((pad_e,), N, jnp.int32)])
    dst = jnp.concatenate(
        [edge_index[1], jnp.full((pad_e,), N, jnp.int32)])
    srcT = src.reshape(NW, EPT_CHUNKS, CHUNK)
    dstT = dst.reshape(NW, EPT_CHUNKS, CHUNK)
    batch_pad = jnp.pad(batch, (0, N_PAD - N),
                        constant_values=G).reshape(N_PAD, 1)

    deg_parts = _deg_sc(dstT)
    hs1, dinv = _prep_tc(x_pad, W1, deg_parts)
    agg1 = _agg_sc(hs1, srcT, dstT)
    hs2 = _mid_tc(agg1, hs1, dinv, b1, g1, be1, W2)
    agg2 = _agg_sc(hs2, srcT, dstT)
    hs3 = _mid_tc(agg2, hs2, dinv, b2, g2, be2, W3)
    agg3 = _agg_sc(hs3, srcT, dstT)
    out = _final_tc(agg3, hs3, dinv, b3, batch_pad, Wc, bc)
    return out
